# baseline (speedup 1.0000x reference)
import functools

import jax
import jax.numpy as jnp
from jax import lax
from jax.experimental import pallas as pl
from jax.experimental.pallas import tpu as pltpu
from jax.experimental.pallas import tpu_sc as plsc

N = 10000
E = 160000
IN_DIM = 256
NCLS = 40
F = 48
NLAYER = 4
NC, NS = 2, 16
NW = NC * NS
B = 128
EPW = E // NW
NB = -(-EPW // B)
PAD_ROW = N
NPAD = 10240
NACC = NPAD
RPS = NPAD // NS
DEGW = 16
NBUF = 4
NBX = NB + NBUF
NOUT = NB // NBUF


def _mesh():
    return plsc.VectorSubcoreMesh(core_axis_name="c", subcore_axis_name="s")


@functools.partial(
    pl.kernel,
    out_type=jax.ShapeDtypeStruct((NC, NPAD, DEGW), jnp.float32),
    mesh=_mesh(),
    scratch_types=[
        pltpu.VMEM((NBX, B), jnp.int32),
        pltpu.VMEM((B, DEGW), jnp.float32),
        pltpu.VMEM_SHARED((NACC, DEGW), jnp.float32),
    ],
    compiler_params=pltpu.CompilerParams(use_tc_tiling_on_sc=False),
)
def _deg_sc(dstb, ones_rows, zero_rows, pdeg, dst_v, ones_v, acc):
    c = lax.axis_index("c")
    s = lax.axis_index("s")
    w = c * NS + s
    pltpu.sync_copy(zero_rows, acc.at[pl.ds(s * RPS, RPS)])
    pltpu.sync_copy(ones_rows, ones_v)
    pltpu.sync_copy(dstb.at[w], dst_v)
    plsc.subcore_barrier()

    def body(j, carry):
        pltpu.sync_copy(ones_v, acc.at[dst_v.at[j]], add=True)
        return carry

    lax.fori_loop(0, NB, body, 0)
    plsc.subcore_barrier()
    pltpu.sync_copy(acc.at[pl.ds(s * RPS, RPS)], pdeg.at[c, pl.ds(s * RPS, RPS)])


@functools.partial(
    pl.kernel,
    out_type=jax.ShapeDtypeStruct((NC, NPAD, F), jnp.float32),
    mesh=_mesh(),
    scratch_types=(
        [pltpu.VMEM((NBX, B), jnp.int32),
         pltpu.VMEM((NBX, B), jnp.int32)]
        + [pltpu.VMEM((B, F), jnp.float32) for _ in range(NBUF)]
        + [pltpu.VMEM_SHARED((NACC, F), jnp.float32)]
        + [pltpu.SemaphoreType.DMA for _ in range(2 * NBUF)]
    ),
    compiler_params=pltpu.CompilerParams(use_tc_tiling_on_sc=False),
)
def _sa_sc(g, srcb, dstb, zero_rows, p, src_v, dst_v, *bufs_and_sems):
    bufs = bufs_and_sems[:NBUF]
    acc = bufs_and_sems[NBUF]
    gsem = bufs_and_sems[NBUF + 1:NBUF + 1 + NBUF]
    ssem = bufs_and_sems[NBUF + 1 + NBUF:]
    c = lax.axis_index("c")
    s = lax.axis_index("s")
    w = c * NS + s
    pltpu.sync_copy(zero_rows, acc.at[pl.ds(s * RPS, RPS)])
    pltpu.sync_copy(srcb.at[w], src_v)
    pltpu.sync_copy(dstb.at[w], dst_v)
    plsc.subcore_barrier()

    for b in range(NBUF):
        pltpu.async_copy(g.at[src_v.at[b]], bufs[b], gsem[b])

    def outer(o, carry):
        for b in range(NBUF):
            j = o * NBUF + b
            pltpu.make_async_copy(g.at[src_v.at[j]], bufs[b], gsem[b]).wait()
            pltpu.async_copy(bufs[b], acc.at[dst_v.at[j]], ssem[b], add=True)
        for b in range(NBUF):
            j = o * NBUF + b
            pltpu.make_async_copy(bufs[b], acc.at[dst_v.at[j]], ssem[b]).wait()
            pltpu.async_copy(g.at[src_v.at[j + NBUF]], bufs[b], gsem[b])
        return carry

    lax.fori_loop(0, NOUT, outer, 0)
    for b in range(NBUF):
        pltpu.make_async_copy(g.at[src_v.at[0]], bufs[b], gsem[b]).wait()
    plsc.subcore_barrier()
    pltpu.sync_copy(acc.at[pl.ds(s * RPS, RPS)], p.at[c, pl.ds(s * RPS, RPS)])


RZ = 1000


def _z_body(x_ref, wT_ref, b_ref, qT_ref, pdeg_ref,
            z0_ref, z1_ref, z2_ref, z3_ref, g4_ref, dinv_ref, dinv2_ref):
    h0 = jnp.maximum(
        jnp.dot(x_ref[...], wT_ref[...], preferred_element_type=jnp.float32)
        + b_ref[...], 0.0)
    z = jnp.dot(h0, qT_ref[...], preferred_element_type=jnp.float32)
    deg = 1.0 + pdeg_ref[0, :, 0:1] + pdeg_ref[1, :, 0:1]
    dinv = lax.rsqrt(deg)
    z0_ref[...] = z[:, 0:F]
    z1_ref[...] = z[:, F:2 * F] * dinv
    z2_ref[...] = z[:, 2 * F:3 * F] * dinv
    z3_ref[...] = z[:, 3 * F:4 * F] * dinv
    g4_ref[...] = z[:, 4 * F:5 * F] * dinv
    dinv_ref[...] = dinv
    dinv2_ref[...] = 1.0 / deg


def _z_tc(x, thetaT, theta_b2, qT, pdeg):
    shp = jax.ShapeDtypeStruct((N, F), jnp.float32)
    shp1 = jax.ShapeDtypeStruct((N, 1), jnp.float32)
    return pl.pallas_call(
        _z_body,
        grid=(N // RZ,),
        in_specs=[
            pl.BlockSpec((RZ, IN_DIM), lambda i: (i, 0)),
            pl.BlockSpec((IN_DIM, IN_DIM), lambda i: (0, 0)),
            pl.BlockSpec((1, IN_DIM), lambda i: (0, 0)),
            pl.BlockSpec((IN_DIM, 5 * F), lambda i: (0, 0)),
            pl.BlockSpec((NC, RZ, DEGW), lambda i: (0, i, 0)),
        ],
        out_specs=[
            pl.BlockSpec((RZ, F), lambda i: (i, 0)),
            pl.BlockSpec((RZ, F), lambda i: (i, 0)),
            pl.BlockSpec((RZ, F), lambda i: (i, 0)),
            pl.BlockSpec((RZ, F), lambda i: (i, 0)),
            pl.BlockSpec((RZ, F), lambda i: (i, 0)),
            pl.BlockSpec((RZ, 1), lambda i: (i, 0)),
            pl.BlockSpec((RZ, 1), lambda i: (i, 0)),
        ],
        out_shape=[shp, shp, shp, shp, shp, shp1, shp1],
    )(x, thetaT, theta_b2, qT, pdeg)


RC = 2000


def _comb_body(p_ref, g_ref, zh_ref, dinv2_ref, out_ref):
    out_ref[...] = (dinv2_ref[...] * (p_ref[0] + p_ref[1] + g_ref[...])
                    + zh_ref[...])


def _comb_tc(p, g, zh, dinv2):
    return pl.pallas_call(
        _comb_body,
        grid=(N // RC,),
        in_specs=[
            pl.BlockSpec((NC, RC, F), lambda i: (0, i, 0)),
            pl.BlockSpec((RC, F), lambda i: (i, 0)),
            pl.BlockSpec((RC, F), lambda i: (i, 0)),
            pl.BlockSpec((RC, 1), lambda i: (i, 0)),
        ],
        out_specs=pl.BlockSpec((RC, F), lambda i: (i, 0)),
        out_shape=jax.ShapeDtypeStruct((N, F), jnp.float32),
    )(p, g, zh, dinv2)


def _fin_body(p_ref, g_ref, z0_ref, dinv_ref, pb_ref, out_ref):
    t = dinv_ref[...] * (p_ref[0] + p_ref[1] + g_ref[...]) + z0_ref[...]
    logits = t[:, :NCLS] + pb_ref[...]
    m = jnp.max(logits, axis=1, keepdims=True)
    lse = jnp.log(jnp.sum(jnp.exp(logits - m), axis=1, keepdims=True))
    out_ref[...] = logits - m - lse


def _fin_tc(p, g, z0, dinv, pb2):
    return pl.pallas_call(
        _fin_body,
        grid=(N // RC,),
        in_specs=[
            pl.BlockSpec((NC, RC, F), lambda i: (0, i, 0)),
            pl.BlockSpec((RC, F), lambda i: (i, 0)),
            pl.BlockSpec((RC, F), lambda i: (i, 0)),
            pl.BlockSpec((RC, 1), lambda i: (i, 0)),
            pl.BlockSpec((1, NCLS), lambda i: (0, 0)),
        ],
        out_specs=pl.BlockSpec((RC, NCLS), lambda i: (i, 0)),
        out_shape=jax.ShapeDtypeStruct((N, NCLS), jnp.float32),
    )(p, g, z0, dinv, pb2)


def kernel(x, edge_index, lambdas, theta_W, theta_b, pred_W, pred_b):
    src = edge_index[0].astype(jnp.int32)
    dst = edge_index[1].astype(jnp.int32)
    padn = NBX * B - EPW
    srcb = jnp.concatenate(
        [src.reshape(NW, EPW), jnp.zeros((NW, padn), jnp.int32)],
        axis=1).reshape(NW, NBX, B)
    dstb = jnp.concatenate(
        [dst.reshape(NW, EPW), jnp.full((NW, padn), PAD_ROW, jnp.int32)],
        axis=1).reshape(NW, NBX, B)

    zero_f = jnp.zeros((RPS, F), jnp.float32)
    zero_d = jnp.zeros((RPS, DEGW), jnp.float32)
    ones_d = jnp.ones((B, DEGW), jnp.float32)

    pdeg = _deg_sc(dstb, ones_d, zero_d)

    lam = 1.0 + jax.nn.relu(lambdas)
    alpha = (2.0 * lam - 2.0) / lam
    beta = 2.0 / lam
    rows = [jnp.zeros((NLAYER + 1,), jnp.float32).at[0].set(1.0)]
    for k in range(NLAYER):
        prev = rows[-1]
        shifted = jnp.concatenate([jnp.zeros((1,), jnp.float32), prev[:-1]])
        rows.append(alpha[k] * prev + beta[k] * shifted)
    cmat = jnp.stack(rows[1:])
    Pk = pred_W.reshape(NCLS, NLAYER, IN_DIM)
    Q = jnp.einsum("kj,ckf->jcf", cmat, Pk)
    qT = jnp.pad(Q, ((0, 0), (0, F - NCLS), (0, 0))).reshape(5 * F, IN_DIM).T

    z0, z1, z2, z3, g, dinv, dinv2 = _z_tc(
        x, theta_W.T, theta_b.reshape(1, IN_DIM), qT, pdeg)

    zh = [None, z1, z2, z3]
    for j in range(NLAYER - 1, 0, -1):
        p = _sa_sc(g, srcb, dstb, zero_f)
        g = _comb_tc(p, g, zh[j], dinv2)
    p = _sa_sc(g, srcb, dstb, zero_f)
    return _fin_tc(p, g, z0, dinv, pred_b.reshape(1, NCLS))


# speedup vs baseline: 1.4255x; 1.4255x over previous
import functools

import jax
import jax.numpy as jnp
from jax import lax
from jax.experimental import pallas as pl
from jax.experimental.pallas import tpu as pltpu
from jax.experimental.pallas import tpu_sc as plsc

N = 10000
E = 160000
IN_DIM = 256
NCLS = 40
F = 48
NLAYER = 4
NC, NS = 2, 16
NW = NC * NS
B = 128
EPW = E // NW
NB = -(-EPW // B)
PAD_ROW = N
NPAD = 10240
NACC = NPAD
RPS = NPAD // NS
DEGW = 16
NBUF = 2
NBX = NB + NBUF
NOUT = NB // NBUF


def _mesh():
    return plsc.VectorSubcoreMesh(core_axis_name="c", subcore_axis_name="s")


@functools.partial(
    pl.kernel,
    out_type=jax.ShapeDtypeStruct((NC, NPAD, DEGW), jnp.float32),
    mesh=_mesh(),
    scratch_types=[
        pltpu.VMEM((NBX, B), jnp.int32),
        pltpu.VMEM((B, DEGW), jnp.float32),
        pltpu.VMEM_SHARED((NACC, DEGW), jnp.float32),
    ],
    compiler_params=pltpu.CompilerParams(use_tc_tiling_on_sc=False),
)
def _deg_sc(dstb, ones_rows, zero_rows, pdeg, dst_v, ones_v, acc):
    c = lax.axis_index("c")
    s = lax.axis_index("s")
    w = c * NS + s
    pltpu.sync_copy(zero_rows, acc.at[pl.ds(s * RPS, RPS)])
    pltpu.sync_copy(ones_rows, ones_v)
    pltpu.sync_copy(dstb.at[w], dst_v)
    plsc.subcore_barrier()

    def body(j, carry):
        pltpu.sync_copy(ones_v, acc.at[dst_v.at[j]], add=True)
        return carry

    lax.fori_loop(0, NB, body, 0)
    plsc.subcore_barrier()
    pltpu.sync_copy(acc.at[pl.ds(s * RPS, RPS)], pdeg.at[c, pl.ds(s * RPS, RPS)])


@functools.partial(
    pl.kernel,
    out_type=jax.ShapeDtypeStruct((NC, NPAD, F), jnp.float32),
    mesh=_mesh(),
    scratch_types=(
        [pltpu.VMEM((NBX, B), jnp.int32),
         pltpu.VMEM((NBX, B), jnp.int32)]
        + [pltpu.VMEM((B, F), jnp.float32) for _ in range(NBUF)]
        + [pltpu.VMEM_SHARED((NACC, F), jnp.float32)]
        + [pltpu.SemaphoreType.DMA for _ in range(2 * NBUF)]
    ),
    compiler_params=pltpu.CompilerParams(use_tc_tiling_on_sc=False),
)
def _sa_sc(g, srcb, dstb, zero_rows, p, src_v, dst_v, *bufs_and_sems):
    bufs = bufs_and_sems[:NBUF]
    acc = bufs_and_sems[NBUF]
    gsem = bufs_and_sems[NBUF + 1:NBUF + 1 + NBUF]
    ssem = bufs_and_sems[NBUF + 1 + NBUF:]
    c = lax.axis_index("c")
    s = lax.axis_index("s")
    w = c * NS + s
    pltpu.sync_copy(zero_rows, acc.at[pl.ds(s * RPS, RPS)])
    pltpu.sync_copy(srcb.at[w], src_v)
    pltpu.sync_copy(dstb.at[w], dst_v)
    plsc.subcore_barrier()

    for b in range(NBUF):
        pltpu.async_copy(g.at[src_v.at[b]], bufs[b], gsem[b])

    def outer(o, carry):
        for b in range(NBUF):
            j = o * NBUF + b
            pltpu.make_async_copy(g.at[src_v.at[j]], bufs[b], gsem[b]).wait()
            pltpu.async_copy(bufs[b], acc.at[dst_v.at[j]], ssem[b], add=True)
        for b in range(NBUF):
            j = o * NBUF + b
            pltpu.make_async_copy(bufs[b], acc.at[dst_v.at[j]], ssem[b]).wait()
            pltpu.async_copy(g.at[src_v.at[j + NBUF]], bufs[b], gsem[b])
        return carry

    lax.fori_loop(0, NOUT, outer, 0)
    for b in range(NBUF):
        pltpu.make_async_copy(g.at[src_v.at[0]], bufs[b], gsem[b]).wait()
    plsc.subcore_barrier()
    pltpu.sync_copy(acc.at[pl.ds(s * RPS, RPS)], p.at[c, pl.ds(s * RPS, RPS)])


RZ = 1000


def _z_body(x_ref, wT_ref, b_ref, qT_ref, pdeg_ref,
            z0_ref, z1_ref, z2_ref, z3_ref, g4_ref, dinv_ref, dinv2_ref):
    h0 = jnp.maximum(
        jnp.dot(x_ref[...], wT_ref[...], preferred_element_type=jnp.float32)
        + b_ref[...], 0.0)
    z = jnp.dot(h0, qT_ref[...], preferred_element_type=jnp.float32)
    deg = 1.0 + pdeg_ref[0, :, 0:1] + pdeg_ref[1, :, 0:1]
    dinv = lax.rsqrt(deg)
    z0_ref[...] = z[:, 0:F]
    z1_ref[...] = z[:, F:2 * F] * dinv
    z2_ref[...] = z[:, 2 * F:3 * F] * dinv
    z3_ref[...] = z[:, 3 * F:4 * F] * dinv
    g4_ref[...] = z[:, 4 * F:5 * F] * dinv
    dinv_ref[...] = dinv
    dinv2_ref[...] = 1.0 / deg


def _z_tc(x, thetaT, theta_b2, qT, pdeg):
    shp = jax.ShapeDtypeStruct((N, F), jnp.float32)
    shp1 = jax.ShapeDtypeStruct((N, 1), jnp.float32)
    return pl.pallas_call(
        _z_body,
        grid=(N // RZ,),
        in_specs=[
            pl.BlockSpec((RZ, IN_DIM), lambda i: (i, 0)),
            pl.BlockSpec((IN_DIM, IN_DIM), lambda i: (0, 0)),
            pl.BlockSpec((1, IN_DIM), lambda i: (0, 0)),
            pl.BlockSpec((IN_DIM, 5 * F), lambda i: (0, 0)),
            pl.BlockSpec((NC, RZ, DEGW), lambda i: (0, i, 0)),
        ],
        out_specs=[
            pl.BlockSpec((RZ, F), lambda i: (i, 0)),
            pl.BlockSpec((RZ, F), lambda i: (i, 0)),
            pl.BlockSpec((RZ, F), lambda i: (i, 0)),
            pl.BlockSpec((RZ, F), lambda i: (i, 0)),
            pl.BlockSpec((RZ, F), lambda i: (i, 0)),
            pl.BlockSpec((RZ, 1), lambda i: (i, 0)),
            pl.BlockSpec((RZ, 1), lambda i: (i, 0)),
        ],
        out_shape=[shp, shp, shp, shp, shp, shp1, shp1],
    )(x, thetaT, theta_b2, qT, pdeg)


RC = 2000


def _comb_body(p_ref, g_ref, zh_ref, dinv2_ref, out_ref):
    out_ref[...] = (dinv2_ref[...] * (p_ref[0] + p_ref[1] + g_ref[...])
                    + zh_ref[...])


def _comb_tc(p, g, zh, dinv2):
    return pl.pallas_call(
        _comb_body,
        grid=(N // RC,),
        in_specs=[
            pl.BlockSpec((NC, RC, F), lambda i: (0, i, 0)),
            pl.BlockSpec((RC, F), lambda i: (i, 0)),
            pl.BlockSpec((RC, F), lambda i: (i, 0)),
            pl.BlockSpec((RC, 1), lambda i: (i, 0)),
        ],
        out_specs=pl.BlockSpec((RC, F), lambda i: (i, 0)),
        out_shape=jax.ShapeDtypeStruct((N, F), jnp.float32),
    )(p, g, zh, dinv2)


def _fin_body(p_ref, g_ref, z0_ref, dinv_ref, pb_ref, out_ref):
    t = dinv_ref[...] * (p_ref[0] + p_ref[1] + g_ref[...]) + z0_ref[...]
    logits = t[:, :NCLS] + pb_ref[...]
    m = jnp.max(logits, axis=1, keepdims=True)
    lse = jnp.log(jnp.sum(jnp.exp(logits - m), axis=1, keepdims=True))
    out_ref[...] = logits - m - lse


def _fin_tc(p, g, z0, dinv, pb2):
    return pl.pallas_call(
        _fin_body,
        grid=(N // RC,),
        in_specs=[
            pl.BlockSpec((NC, RC, F), lambda i: (0, i, 0)),
            pl.BlockSpec((RC, F), lambda i: (i, 0)),
            pl.BlockSpec((RC, F), lambda i: (i, 0)),
            pl.BlockSpec((RC, 1), lambda i: (i, 0)),
            pl.BlockSpec((1, NCLS), lambda i: (0, 0)),
        ],
        out_specs=pl.BlockSpec((RC, NCLS), lambda i: (i, 0)),
        out_shape=jax.ShapeDtypeStruct((N, NCLS), jnp.float32),
    )(p, g, z0, dinv, pb2)


def kernel(x, edge_index, lambdas, theta_W, theta_b, pred_W, pred_b):
    src = edge_index[0].astype(jnp.int32)
    dst = edge_index[1].astype(jnp.int32)
    padn = NBX * B - EPW
    srcb = jnp.concatenate(
        [src.reshape(NW, EPW), jnp.zeros((NW, padn), jnp.int32)],
        axis=1).reshape(NW, NBX, B)
    dstb = jnp.concatenate(
        [dst.reshape(NW, EPW), jnp.full((NW, padn), PAD_ROW, jnp.int32)],
        axis=1).reshape(NW, NBX, B)

    zero_f = jnp.zeros((RPS, F), jnp.float32)
    zero_d = jnp.zeros((RPS, DEGW), jnp.float32)
    ones_d = jnp.ones((B, DEGW), jnp.float32)

    pdeg = _deg_sc(dstb, ones_d, zero_d)

    lam = 1.0 + jax.nn.relu(lambdas)
    alpha = (2.0 * lam - 2.0) / lam
    beta = 2.0 / lam
    rows = [jnp.zeros((NLAYER + 1,), jnp.float32).at[0].set(1.0)]
    for k in range(NLAYER):
        prev = rows[-1]
        shifted = jnp.concatenate([jnp.zeros((1,), jnp.float32), prev[:-1]])
        rows.append(alpha[k] * prev + beta[k] * shifted)
    cmat = jnp.stack(rows[1:])
    Pk = pred_W.reshape(NCLS, NLAYER, IN_DIM)
    Q = jnp.einsum("kj,ckf->jcf", cmat, Pk)
    qT = jnp.pad(Q, ((0, 0), (0, F - NCLS), (0, 0))).reshape(5 * F, IN_DIM).T

    z0, z1, z2, z3, g, dinv, dinv2 = _z_tc(
        x, theta_W.T, theta_b.reshape(1, IN_DIM), qT, pdeg)

    zh = [None, z1, z2, z3]
    for j in range(NLAYER - 1, 0, -1):
        p = _sa_sc(g, srcb, dstb, zero_f)
        g = _comb_tc(p, g, zh[j], dinv2)
    p = _sa_sc(g, srcb, dstb, zero_f)
    return _fin_tc(p, g, z0, dinv, pred_b.reshape(1, NCLS))


# speedup vs baseline: 1.7751x; 1.2453x over previous
import functools

import jax
import jax.numpy as jnp
from jax import lax
from jax.experimental import pallas as pl
from jax.experimental.pallas import tpu as pltpu
from jax.experimental.pallas import tpu_sc as plsc

N = 10000
E = 160000
IN_DIM = 256
NCLS = 40
F = 48
NLAYER = 4
NC, NS = 2, 16
NW = NC * NS
B = 128
EPW = E // NW
NB = -(-EPW // B)
PAD_ROW = N
NPAD = 10240
NACC = NPAD
RPS = NPAD // NS
DEGW = 16
NBUF = 2
NBX = NB + NBUF
NOUT = NB // NBUF


def _mesh():
    return plsc.VectorSubcoreMesh(core_axis_name="c", subcore_axis_name="s")


@functools.partial(
    pl.kernel,
    out_type=jax.ShapeDtypeStruct((NC, NPAD, DEGW), jnp.float32),
    mesh=_mesh(),
    scratch_types=[
        pltpu.VMEM((NBX, B), jnp.int32),
        pltpu.VMEM((B, DEGW), jnp.float32),
        pltpu.VMEM_SHARED((NACC, DEGW), jnp.float32),
    ],
    compiler_params=pltpu.CompilerParams(use_tc_tiling_on_sc=False),
)
def _deg_sc(dstb, ones_rows, zero_rows, pdeg, dst_v, ones_v, acc):
    c = lax.axis_index("c")
    s = lax.axis_index("s")
    w = c * NS + s
    pltpu.sync_copy(zero_rows, acc.at[pl.ds(s * RPS, RPS)])
    pltpu.sync_copy(ones_rows, ones_v)
    pltpu.sync_copy(dstb.at[w], dst_v)
    plsc.subcore_barrier()

    def body(j, carry):
        pltpu.sync_copy(ones_v, acc.at[dst_v.at[j]], add=True)
        return carry

    lax.fori_loop(0, NB, body, 0)
    plsc.subcore_barrier()
    pltpu.sync_copy(acc.at[pl.ds(s * RPS, RPS)], pdeg.at[c, pl.ds(s * RPS, RPS)])


@functools.partial(
    pl.kernel,
    out_type=jax.ShapeDtypeStruct((NC, NPAD, F), jnp.float32),
    mesh=_mesh(),
    scratch_types=(
        [pltpu.VMEM((NBX, B), jnp.int32),
         pltpu.VMEM((NBX, B), jnp.int32)]
        + [pltpu.VMEM((B, F), jnp.float32) for _ in range(NBUF)]
        + [pltpu.VMEM_SHARED((NACC, F), jnp.float32)]
        + [pltpu.SemaphoreType.DMA for _ in range(2 * NBUF)]
    ),
    compiler_params=pltpu.CompilerParams(use_tc_tiling_on_sc=False),
)
def _sa_sc(g, srcb, dstb, zero_rows, p, src_v, dst_v, *bufs_and_sems):
    bufs = bufs_and_sems[:NBUF]
    acc = bufs_and_sems[NBUF]
    gsem = bufs_and_sems[NBUF + 1:NBUF + 1 + NBUF]
    ssem = bufs_and_sems[NBUF + 1 + NBUF:]
    c = lax.axis_index("c")
    s = lax.axis_index("s")
    w = c * NS + s
    pltpu.sync_copy(zero_rows, acc.at[pl.ds(s * RPS, RPS)])
    pltpu.sync_copy(srcb.at[w], src_v)
    pltpu.sync_copy(dstb.at[w], dst_v)
    plsc.subcore_barrier()

    pltpu.async_copy(g.at[src_v.at[0]], bufs[0], gsem[0])

    def outer(o, carry):
        for b in range(NBUF):
            j = o * NBUF + b
            pltpu.make_async_copy(g.at[src_v.at[j]], bufs[b], gsem[b]).wait()
            nb = (b + 1) % NBUF
            pltpu.async_copy(g.at[src_v.at[j + 1]], bufs[nb], gsem[nb])
            pltpu.sync_copy(bufs[b], acc.at[dst_v.at[j]], add=True)
        return carry

    lax.fori_loop(0, NOUT, outer, 0)
    pltpu.make_async_copy(g.at[src_v.at[0]], bufs[0], gsem[0]).wait()
    plsc.subcore_barrier()
    pltpu.sync_copy(acc.at[pl.ds(s * RPS, RPS)], p.at[c, pl.ds(s * RPS, RPS)])


RZ = 1000


def _z_body(x_ref, wT_ref, b_ref, qT_ref, pdeg_ref,
            z0_ref, z1_ref, z2_ref, z3_ref, g4_ref, dinv_ref, dinv2_ref):
    h0 = jnp.maximum(
        jnp.dot(x_ref[...], wT_ref[...], preferred_element_type=jnp.float32)
        + b_ref[...], 0.0)
    z = jnp.dot(h0, qT_ref[...], preferred_element_type=jnp.float32)
    deg = 1.0 + pdeg_ref[0, :, 0:1] + pdeg_ref[1, :, 0:1]
    dinv = lax.rsqrt(deg)
    z0_ref[...] = z[:, 0:F]
    z1_ref[...] = z[:, F:2 * F] * dinv
    z2_ref[...] = z[:, 2 * F:3 * F] * dinv
    z3_ref[...] = z[:, 3 * F:4 * F] * dinv
    g4_ref[...] = z[:, 4 * F:5 * F] * dinv
    dinv_ref[...] = dinv
    dinv2_ref[...] = 1.0 / deg


def _z_tc(x, thetaT, theta_b2, qT, pdeg):
    shp = jax.ShapeDtypeStruct((N, F), jnp.float32)
    shp1 = jax.ShapeDtypeStruct((N, 1), jnp.float32)
    return pl.pallas_call(
        _z_body,
        grid=(N // RZ,),
        in_specs=[
            pl.BlockSpec((RZ, IN_DIM), lambda i: (i, 0)),
            pl.BlockSpec((IN_DIM, IN_DIM), lambda i: (0, 0)),
            pl.BlockSpec((1, IN_DIM), lambda i: (0, 0)),
            pl.BlockSpec((IN_DIM, 5 * F), lambda i: (0, 0)),
            pl.BlockSpec((NC, RZ, DEGW), lambda i: (0, i, 0)),
        ],
        out_specs=[
            pl.BlockSpec((RZ, F), lambda i: (i, 0)),
            pl.BlockSpec((RZ, F), lambda i: (i, 0)),
            pl.BlockSpec((RZ, F), lambda i: (i, 0)),
            pl.BlockSpec((RZ, F), lambda i: (i, 0)),
            pl.BlockSpec((RZ, F), lambda i: (i, 0)),
            pl.BlockSpec((RZ, 1), lambda i: (i, 0)),
            pl.BlockSpec((RZ, 1), lambda i: (i, 0)),
        ],
        out_shape=[shp, shp, shp, shp, shp, shp1, shp1],
    )(x, thetaT, theta_b2, qT, pdeg)


RC = 2000


def _comb_body(p_ref, g_ref, zh_ref, dinv2_ref, out_ref):
    out_ref[...] = (dinv2_ref[...] * (p_ref[0] + p_ref[1] + g_ref[...])
                    + zh_ref[...])


def _comb_tc(p, g, zh, dinv2):
    return pl.pallas_call(
        _comb_body,
        grid=(N // RC,),
        in_specs=[
            pl.BlockSpec((NC, RC, F), lambda i: (0, i, 0)),
            pl.BlockSpec((RC, F), lambda i: (i, 0)),
            pl.BlockSpec((RC, F), lambda i: (i, 0)),
            pl.BlockSpec((RC, 1), lambda i: (i, 0)),
        ],
        out_specs=pl.BlockSpec((RC, F), lambda i: (i, 0)),
        out_shape=jax.ShapeDtypeStruct((N, F), jnp.float32),
    )(p, g, zh, dinv2)


def _fin_body(p_ref, g_ref, z0_ref, dinv_ref, pb_ref, out_ref):
    t = dinv_ref[...] * (p_ref[0] + p_ref[1] + g_ref[...]) + z0_ref[...]
    logits = t[:, :NCLS] + pb_ref[...]
    m = jnp.max(logits, axis=1, keepdims=True)
    lse = jnp.log(jnp.sum(jnp.exp(logits - m), axis=1, keepdims=True))
    out_ref[...] = logits - m - lse


def _fin_tc(p, g, z0, dinv, pb2):
    return pl.pallas_call(
        _fin_body,
        grid=(N // RC,),
        in_specs=[
            pl.BlockSpec((NC, RC, F), lambda i: (0, i, 0)),
            pl.BlockSpec((RC, F), lambda i: (i, 0)),
            pl.BlockSpec((RC, F), lambda i: (i, 0)),
            pl.BlockSpec((RC, 1), lambda i: (i, 0)),
            pl.BlockSpec((1, NCLS), lambda i: (0, 0)),
        ],
        out_specs=pl.BlockSpec((RC, NCLS), lambda i: (i, 0)),
        out_shape=jax.ShapeDtypeStruct((N, NCLS), jnp.float32),
    )(p, g, z0, dinv, pb2)


def kernel(x, edge_index, lambdas, theta_W, theta_b, pred_W, pred_b):
    src = edge_index[0].astype(jnp.int32)
    dst = edge_index[1].astype(jnp.int32)
    padn = NBX * B - EPW
    srcb = jnp.concatenate(
        [src.reshape(NW, EPW), jnp.zeros((NW, padn), jnp.int32)],
        axis=1).reshape(NW, NBX, B)
    dstb = jnp.concatenate(
        [dst.reshape(NW, EPW), jnp.full((NW, padn), PAD_ROW, jnp.int32)],
        axis=1).reshape(NW, NBX, B)

    zero_f = jnp.zeros((RPS, F), jnp.float32)
    zero_d = jnp.zeros((RPS, DEGW), jnp.float32)
    ones_d = jnp.ones((B, DEGW), jnp.float32)

    pdeg = _deg_sc(dstb, ones_d, zero_d)

    lam = 1.0 + jax.nn.relu(lambdas)
    alpha = (2.0 * lam - 2.0) / lam
    beta = 2.0 / lam
    rows = [jnp.zeros((NLAYER + 1,), jnp.float32).at[0].set(1.0)]
    for k in range(NLAYER):
        prev = rows[-1]
        shifted = jnp.concatenate([jnp.zeros((1,), jnp.float32), prev[:-1]])
        rows.append(alpha[k] * prev + beta[k] * shifted)
    cmat = jnp.stack(rows[1:])
    Pk = pred_W.reshape(NCLS, NLAYER, IN_DIM)
    Q = jnp.einsum("kj,ckf->jcf", cmat, Pk)
    qT = jnp.pad(Q, ((0, 0), (0, F - NCLS), (0, 0))).reshape(5 * F, IN_DIM).T

    z0, z1, z2, z3, g, dinv, dinv2 = _z_tc(
        x, theta_W.T, theta_b.reshape(1, IN_DIM), qT, pdeg)

    zh = [None, z1, z2, z3]
    for j in range(NLAYER - 1, 0, -1):
        p = _sa_sc(g, srcb, dstb, zero_f)
        g = _comb_tc(p, g, zh[j], dinv2)
    p = _sa_sc(g, srcb, dstb, zero_f)
    return _fin_tc(p, g, z0, dinv, pred_b.reshape(1, NCLS))


# speedup vs baseline: 2.5866x; 1.4571x over previous
import functools

import jax
import jax.numpy as jnp
from jax import lax
from jax.experimental import pallas as pl
from jax.experimental.pallas import tpu as pltpu
from jax.experimental.pallas import tpu_sc as plsc

N = 10000
E = 160000
IN_DIM = 256
NCLS = 40
F = 48
NLAYER = 4
NC, NS = 2, 16
NW = NC * NS
B = 512
EPW = E // NW
NB = -(-EPW // B)
PAD_ROW = N
NPAD = 10240
NACC = NPAD
RPS = NPAD // NS
DEGW = 16
NBX = NB


def _mesh():
    return plsc.VectorSubcoreMesh(core_axis_name="c", subcore_axis_name="s")


@functools.partial(
    pl.kernel,
    out_type=jax.ShapeDtypeStruct((NC, NPAD, DEGW), jnp.float32),
    mesh=_mesh(),
    scratch_types=[
        pltpu.VMEM((NBX, B), jnp.int32),
        pltpu.VMEM((B, DEGW), jnp.float32),
        pltpu.VMEM_SHARED((NACC, DEGW), jnp.float32),
    ],
    compiler_params=pltpu.CompilerParams(use_tc_tiling_on_sc=False),
)
def _deg_sc(dstb, ones_rows, zero_rows, pdeg, dst_v, ones_v, acc):
    c = lax.axis_index("c")
    s = lax.axis_index("s")
    w = c * NS + s
    pltpu.sync_copy(zero_rows, acc.at[pl.ds(s * RPS, RPS)])
    pltpu.sync_copy(ones_rows, ones_v)
    pltpu.sync_copy(dstb.at[w], dst_v)
    plsc.subcore_barrier()

    def body(j, carry):
        pltpu.sync_copy(ones_v, acc.at[dst_v.at[j]], add=True)
        return carry

    lax.fori_loop(0, NB, body, 0)
    plsc.subcore_barrier()
    pltpu.sync_copy(acc.at[pl.ds(s * RPS, RPS)], pdeg.at[c, pl.ds(s * RPS, RPS)])


@functools.partial(
    pl.kernel,
    out_type=jax.ShapeDtypeStruct((NC, NPAD, F), jnp.float32),
    mesh=_mesh(),
    scratch_types=[
        pltpu.VMEM((NBX, B), jnp.int32),
        pltpu.VMEM((NBX, B), jnp.int32),
        pltpu.VMEM((B, F), jnp.float32),
        pltpu.VMEM_SHARED((NACC, F), jnp.float32),
        pltpu.SemaphoreType.DMA,
    ],
    compiler_params=pltpu.CompilerParams(use_tc_tiling_on_sc=False),
)
def _sa_sc(g, srcb, dstb, zero_rows, p, src_v, dst_v, rows_v, acc, sem):
    c = lax.axis_index("c")
    s = lax.axis_index("s")
    w = c * NS + s
    pltpu.sync_copy(zero_rows, acc.at[pl.ds(s * RPS, RPS)])
    pltpu.sync_copy(srcb.at[w], src_v)
    pltpu.sync_copy(dstb.at[w], dst_v)
    plsc.subcore_barrier()

    def body(j, carry):
        pltpu.async_copy(g.at[src_v.at[j]], rows_v, sem).wait()
        pltpu.sync_copy(rows_v, acc.at[dst_v.at[j]], add=True)
        return carry

    lax.fori_loop(0, NB, body, 0)
    plsc.subcore_barrier()
    pltpu.sync_copy(acc.at[pl.ds(s * RPS, RPS)], p.at[c, pl.ds(s * RPS, RPS)])


RZ = 1000


def _z_body(x_ref, wT_ref, b_ref, qT_ref, pdeg_ref,
            z0_ref, z1_ref, z2_ref, z3_ref, g4_ref, dinv_ref, dinv2_ref):
    h0 = jnp.maximum(
        jnp.dot(x_ref[...], wT_ref[...], preferred_element_type=jnp.float32)
        + b_ref[...], 0.0)
    z = jnp.dot(h0, qT_ref[...], preferred_element_type=jnp.float32)
    deg = 1.0 + pdeg_ref[0, :, 0:1] + pdeg_ref[1, :, 0:1]
    dinv = lax.rsqrt(deg)
    z0_ref[...] = z[:, 0:F]
    z1_ref[...] = z[:, F:2 * F] * dinv
    z2_ref[...] = z[:, 2 * F:3 * F] * dinv
    z3_ref[...] = z[:, 3 * F:4 * F] * dinv
    g4_ref[...] = z[:, 4 * F:5 * F] * dinv
    dinv_ref[...] = dinv
    dinv2_ref[...] = 1.0 / deg


def _z_tc(x, thetaT, theta_b2, qT, pdeg):
    shp = jax.ShapeDtypeStruct((N, F), jnp.float32)
    shp1 = jax.ShapeDtypeStruct((N, 1), jnp.float32)
    return pl.pallas_call(
        _z_body,
        grid=(N // RZ,),
        in_specs=[
            pl.BlockSpec((RZ, IN_DIM), lambda i: (i, 0)),
            pl.BlockSpec((IN_DIM, IN_DIM), lambda i: (0, 0)),
            pl.BlockSpec((1, IN_DIM), lambda i: (0, 0)),
            pl.BlockSpec((IN_DIM, 5 * F), lambda i: (0, 0)),
            pl.BlockSpec((NC, RZ, DEGW), lambda i: (0, i, 0)),
        ],
        out_specs=[
            pl.BlockSpec((RZ, F), lambda i: (i, 0)),
            pl.BlockSpec((RZ, F), lambda i: (i, 0)),
            pl.BlockSpec((RZ, F), lambda i: (i, 0)),
            pl.BlockSpec((RZ, F), lambda i: (i, 0)),
            pl.BlockSpec((RZ, F), lambda i: (i, 0)),
            pl.BlockSpec((RZ, 1), lambda i: (i, 0)),
            pl.BlockSpec((RZ, 1), lambda i: (i, 0)),
        ],
        out_shape=[shp, shp, shp, shp, shp, shp1, shp1],
    )(x, thetaT, theta_b2, qT, pdeg)


RC = 2000


def _comb_body(p_ref, g_ref, zh_ref, dinv2_ref, out_ref):
    out_ref[...] = (dinv2_ref[...] * (p_ref[0] + p_ref[1] + g_ref[...])
                    + zh_ref[...])


def _comb_tc(p, g, zh, dinv2):
    return pl.pallas_call(
        _comb_body,
        grid=(N // RC,),
        in_specs=[
            pl.BlockSpec((NC, RC, F), lambda i: (0, i, 0)),
            pl.BlockSpec((RC, F), lambda i: (i, 0)),
            pl.BlockSpec((RC, F), lambda i: (i, 0)),
            pl.BlockSpec((RC, 1), lambda i: (i, 0)),
        ],
        out_specs=pl.BlockSpec((RC, F), lambda i: (i, 0)),
        out_shape=jax.ShapeDtypeStruct((N, F), jnp.float32),
    )(p, g, zh, dinv2)


def _fin_body(p_ref, g_ref, z0_ref, dinv_ref, pb_ref, out_ref):
    t = dinv_ref[...] * (p_ref[0] + p_ref[1] + g_ref[...]) + z0_ref[...]
    logits = t[:, :NCLS] + pb_ref[...]
    m = jnp.max(logits, axis=1, keepdims=True)
    lse = jnp.log(jnp.sum(jnp.exp(logits - m), axis=1, keepdims=True))
    out_ref[...] = logits - m - lse


def _fin_tc(p, g, z0, dinv, pb2):
    return pl.pallas_call(
        _fin_body,
        grid=(N // RC,),
        in_specs=[
            pl.BlockSpec((NC, RC, F), lambda i: (0, i, 0)),
            pl.BlockSpec((RC, F), lambda i: (i, 0)),
            pl.BlockSpec((RC, F), lambda i: (i, 0)),
            pl.BlockSpec((RC, 1), lambda i: (i, 0)),
            pl.BlockSpec((1, NCLS), lambda i: (0, 0)),
        ],
        out_specs=pl.BlockSpec((RC, NCLS), lambda i: (i, 0)),
        out_shape=jax.ShapeDtypeStruct((N, NCLS), jnp.float32),
    )(p, g, z0, dinv, pb2)


def kernel(x, edge_index, lambdas, theta_W, theta_b, pred_W, pred_b):
    src = edge_index[0].astype(jnp.int32)
    dst = edge_index[1].astype(jnp.int32)
    padn = NBX * B - EPW
    srcb = jnp.concatenate(
        [src.reshape(NW, EPW), jnp.zeros((NW, padn), jnp.int32)],
        axis=1).reshape(NW, NBX, B)
    dstb = jnp.concatenate(
        [dst.reshape(NW, EPW), jnp.full((NW, padn), PAD_ROW, jnp.int32)],
        axis=1).reshape(NW, NBX, B)

    zero_f = jnp.zeros((RPS, F), jnp.float32)
    zero_d = jnp.zeros((RPS, DEGW), jnp.float32)
    ones_d = jnp.ones((B, DEGW), jnp.float32)

    pdeg = _deg_sc(dstb, ones_d, zero_d)

    lam = 1.0 + jax.nn.relu(lambdas)
    alpha = (2.0 * lam - 2.0) / lam
    beta = 2.0 / lam
    rows = [jnp.zeros((NLAYER + 1,), jnp.float32).at[0].set(1.0)]
    for k in range(NLAYER):
        prev = rows[-1]
        shifted = jnp.concatenate([jnp.zeros((1,), jnp.float32), prev[:-1]])
        rows.append(alpha[k] * prev + beta[k] * shifted)
    cmat = jnp.stack(rows[1:])
    Pk = pred_W.reshape(NCLS, NLAYER, IN_DIM)
    Q = jnp.einsum("kj,ckf->jcf", cmat, Pk)
    qT = jnp.pad(Q, ((0, 0), (0, F - NCLS), (0, 0))).reshape(5 * F, IN_DIM).T

    z0, z1, z2, z3, g, dinv, dinv2 = _z_tc(
        x, theta_W.T, theta_b.reshape(1, IN_DIM), qT, pdeg)

    zh = [None, z1, z2, z3]
    for j in range(NLAYER - 1, 0, -1):
        p = _sa_sc(g, srcb, dstb, zero_f)
        g = _comb_tc(p, g, zh[j], dinv2)
    p = _sa_sc(g, srcb, dstb, zero_f)
    return _fin_tc(p, g, z0, dinv, pred_b.reshape(1, NCLS))
